# baseline (speedup 1.0000x reference)
import functools

import jax
import jax.numpy as jnp
from jax import lax
from jax.experimental import pallas as pl
from jax.experimental.pallas import tpu as pltpu
from jax.experimental.pallas import tpu_sc as plsc

_NC = 2
_NS = 16
_CH = 128


def _sc_gather(link_tab128, link_ids, dom_tab128, domain_ids):
    B = link_ids.shape[0]
    W = link_tab128.shape[1]
    nw = _NC * _NS
    bpw = B // nw
    nch = bpw // _CH
    mesh = plsc.VectorSubcoreMesh(
        core_axis_name="c", subcore_axis_name="s",
        num_cores=_NC, num_subcores=_NS)

    @functools.partial(
        pl.kernel,
        out_type=[
            jax.ShapeDtypeStruct((B, W), jnp.float32),
            jax.ShapeDtypeStruct((B, W), jnp.float32),
        ],
        mesh=mesh,
        scratch_types=[
            pltpu.VMEM((nch, _CH), jnp.int32),
            pltpu.VMEM((2, _CH, W), jnp.float32),
            pltpu.VMEM((nch, _CH), jnp.int32),
            pltpu.VMEM((2, _CH, W), jnp.float32),
            pltpu.SemaphoreType.DMA,
            pltpu.SemaphoreType.DMA,
        ],
    )
    def gather_kernel(ltab_hbm, lids_hbm, dtab_hbm, dids_hbm,
                      lout_hbm, dout_hbm,
                      lidx_v, lbuf_v, didx_v, dbuf_v, lsem, dsem):
        wid = lax.axis_index("s") * _NC + lax.axis_index("c")
        base = wid * bpw
        for k in range(nch):
            pltpu.sync_copy(lids_hbm.at[pl.ds(base + k * _CH, _CH)],
                            lidx_v.at[k])
            pltpu.sync_copy(dids_hbm.at[pl.ds(base + k * _CH, _CH)],
                            didx_v.at[k])
        lcps = [None] * nch
        dcps = [None] * nch
        lcps[0] = pltpu.async_copy(ltab_hbm.at[lidx_v.at[0]],
                                   lbuf_v.at[0], lsem)
        dcps[0] = pltpu.async_copy(dtab_hbm.at[didx_v.at[0]],
                                   dbuf_v.at[0], dsem)
        for k in range(nch):
            if k + 1 < nch:
                lcps[k + 1] = pltpu.async_copy(
                    ltab_hbm.at[lidx_v.at[k + 1]],
                    lbuf_v.at[(k + 1) % 2], lsem)
                dcps[k + 1] = pltpu.async_copy(
                    dtab_hbm.at[didx_v.at[k + 1]],
                    dbuf_v.at[(k + 1) % 2], dsem)
            lcps[k].wait()
            pltpu.sync_copy(lbuf_v.at[k % 2],
                            lout_hbm.at[pl.ds(base + k * _CH, _CH)])
            dcps[k].wait()
            pltpu.sync_copy(dbuf_v.at[k % 2],
                            dout_hbm.at[pl.ds(base + k * _CH, _CH)])

    return gather_kernel(link_tab128, link_ids, dom_tab128, domain_ids)


def _mlp_body(t_ref, le_ref, de_ref, wt_ref, bt_ref,
              w1a_ref, w1b_ref, w1c_ref, b1_ref, w2_ref, b2_ref, o_ref):
    t = jnp.dot(t_ref[...], wt_ref[...], preferred_element_type=jnp.float32)
    t = t + bt_ref[...]
    h = (jnp.dot(t, w1b_ref[...], preferred_element_type=jnp.float32)
         + jnp.dot(le_ref[..., :w1a_ref.shape[0]], w1a_ref[...],
                   preferred_element_type=jnp.float32)
         + jnp.dot(de_ref[..., :w1c_ref.shape[0]], w1c_ref[...],
                   preferred_element_type=jnp.float32)
         + b1_ref[...])
    h = jnp.maximum(h, 0.0)
    o_ref[...] = jnp.dot(h, w2_ref[...], preferred_element_type=jnp.float32) + b2_ref[...]


def _mlp(text_feats, link_e, dom_e, W_text, b_text,
         W1a, W1b, W1c, b1, W2, b2, interpret=False):
    B = text_feats.shape[0]
    BR = 2048
    grid = (B // BR,)

    def row_block(d):
        return pl.BlockSpec((BR, d), lambda i: (i, 0))

    def full_block(w):
        return pl.BlockSpec(w.shape, lambda i: (0,) * w.ndim)

    return pl.pallas_call(
        _mlp_body,
        grid=grid,
        in_specs=[
            row_block(text_feats.shape[1]),
            row_block(link_e.shape[1]),
            row_block(dom_e.shape[1]),
            full_block(W_text), full_block(b_text),
            full_block(W1a), full_block(W1b), full_block(W1c), full_block(b1),
            full_block(W2), full_block(b2),
        ],
        out_specs=row_block(W2.shape[1]),
        out_shape=jax.ShapeDtypeStruct((B, W2.shape[1]), jnp.float32),
        interpret=interpret,
    )(text_feats, link_e, dom_e, W_text, b_text,
      W1a, W1b, W1c, b1, W2, b2)


def kernel(link_ids, domain_ids, text_feats, link_table, domain_table,
           W_text, b_text, W1, b1, W2, b2):
    DL = link_table.shape[1]
    DT = W_text.shape[1]
    DD = domain_table.shape[1]
    link128 = jnp.pad(link_table, ((0, 0), (0, 128 - DL)))
    dom128 = jnp.pad(domain_table, ((0, 0), (0, 128 - DD)))
    link_e, dom_e = _sc_gather(link128, link_ids, dom128, domain_ids)
    W1a = W1[:DL]
    W1b = W1[DL:DL + DT]
    W1c = W1[DL + DT:]
    return _mlp(text_feats, link_e, dom_e,
                W_text, b_text.reshape(1, -1),
                W1a, W1b, W1c, b1.reshape(1, -1),
                W2, b2.reshape(1, -1))


# speedup vs baseline: 3.9653x; 3.9653x over previous
import functools

import jax
import jax.numpy as jnp
from jax import lax
from jax.experimental import pallas as pl
from jax.experimental.pallas import tpu as pltpu
from jax.experimental.pallas import tpu_sc as plsc

_NC = 2
_NS = 16
_CH = 128


def _sc_gather(link_tab128, link_ids, dom_tab128, domain_ids):
    B = link_ids.shape[0]
    W = link_tab128.shape[1]
    nw = _NC * _NS
    bpw = B // nw
    nch = bpw // _CH
    mesh = plsc.VectorSubcoreMesh(
        core_axis_name="c", subcore_axis_name="s",
        num_cores=_NC, num_subcores=_NS)

    @functools.partial(
        pl.kernel,
        out_type=[
            jax.ShapeDtypeStruct((B, W), jnp.float32),
            jax.ShapeDtypeStruct((B, W), jnp.float32),
        ],
        mesh=mesh,
        scratch_types=[
            pltpu.VMEM((nch, _CH), jnp.int32),
            pltpu.VMEM((2, _CH, W), jnp.float32),
            pltpu.VMEM((nch, _CH), jnp.int32),
            pltpu.VMEM((2, _CH, W), jnp.float32),
            pltpu.SemaphoreType.DMA,
            pltpu.SemaphoreType.DMA,
        ],
    )
    def gather_kernel(ltab_hbm, lids_hbm, dtab_hbm, dids_hbm,
                      lout_hbm, dout_hbm,
                      lidx_v, lbuf_v, didx_v, dbuf_v, lsem, dsem):
        wid = lax.axis_index("s") * _NC + lax.axis_index("c")
        base = wid * bpw
        for k in range(nch):
            pltpu.sync_copy(lids_hbm.at[pl.ds(base + k * _CH, _CH)],
                            lidx_v.at[k])
            pltpu.sync_copy(dids_hbm.at[pl.ds(base + k * _CH, _CH)],
                            didx_v.at[k])
        lcps = [None] * nch
        dcps = [None] * nch
        lcps[0] = pltpu.async_copy(ltab_hbm.at[lidx_v.at[0]],
                                   lbuf_v.at[0], lsem)
        dcps[0] = pltpu.async_copy(dtab_hbm.at[didx_v.at[0]],
                                   dbuf_v.at[0], dsem)
        for k in range(nch):
            if k + 1 < nch:
                lcps[k + 1] = pltpu.async_copy(
                    ltab_hbm.at[lidx_v.at[k + 1]],
                    lbuf_v.at[(k + 1) % 2], lsem)
                dcps[k + 1] = pltpu.async_copy(
                    dtab_hbm.at[didx_v.at[k + 1]],
                    dbuf_v.at[(k + 1) % 2], dsem)
            lcps[k].wait()
            pltpu.sync_copy(lbuf_v.at[k % 2],
                            lout_hbm.at[pl.ds(base + k * _CH, _CH)])
            dcps[k].wait()
            pltpu.sync_copy(dbuf_v.at[k % 2],
                            dout_hbm.at[pl.ds(base + k * _CH, _CH)])

    return gather_kernel(link_tab128, link_ids, dom_tab128, domain_ids)


def _mlp_body(t_ref, le_ref, de_ref, wt_ref, bt_ref,
              w1a_ref, w1b_ref, w1c_ref, b1_ref, w2_ref, b2_ref, o_ref):
    t = jnp.dot(t_ref[...], wt_ref[...], preferred_element_type=jnp.float32)
    t = t + bt_ref[...]
    h = (jnp.dot(t, w1b_ref[...], preferred_element_type=jnp.float32)
         + jnp.dot(le_ref[..., :w1a_ref.shape[0]], w1a_ref[...],
                   preferred_element_type=jnp.float32)
         + jnp.dot(de_ref[..., :w1c_ref.shape[0]], w1c_ref[...],
                   preferred_element_type=jnp.float32)
         + b1_ref[...])
    h = jnp.maximum(h, 0.0)
    o_ref[...] = jnp.dot(h, w2_ref[...], preferred_element_type=jnp.float32) + b2_ref[...]


def _mlp(text_feats, link_e, dom_e, W_text, b_text,
         W1a, W1b, W1c, b1, W2, b2, interpret=False):
    B = text_feats.shape[0]
    BR = 2048
    grid = (B // BR,)

    def row_block(d):
        return pl.BlockSpec((BR, d), lambda i: (i, 0))

    def full_block(w):
        return pl.BlockSpec(w.shape, lambda i: (0,) * w.ndim)

    return pl.pallas_call(
        _mlp_body,
        grid=grid,
        in_specs=[
            row_block(text_feats.shape[1]),
            row_block(link_e.shape[1]),
            row_block(dom_e.shape[1]),
            full_block(W_text), full_block(b_text),
            full_block(W1a), full_block(W1b), full_block(W1c), full_block(b1),
            full_block(W2), full_block(b2),
        ],
        out_specs=row_block(W2.shape[1]),
        out_shape=jax.ShapeDtypeStruct((B, W2.shape[1]), jnp.float32),
        interpret=interpret,
    )(text_feats, link_e, dom_e, W_text, b_text,
      W1a, W1b, W1c, b1, W2, b2)


def kernel(link_ids, domain_ids, text_feats, link_table, domain_table,
           W_text, b_text, W1, b1, W2, b2):
    DL = link_table.shape[1]
    DT = W_text.shape[1]
    DD = domain_table.shape[1]
    pad_l = jnp.eye(DL, 128, dtype=jnp.float32)
    link128 = link_table @ pad_l
    dom128 = jnp.pad(domain_table, ((0, 0), (0, 128 - DD)))
    link_e, dom_e = _sc_gather(link128, link_ids, dom128, domain_ids)
    W1a = W1[:DL]
    W1b = W1[DL:DL + DT]
    W1c = W1[DL + DT:]
    return _mlp(text_feats, link_e, dom_e,
                W_text, b_text.reshape(1, -1),
                W1a, W1b, W1c, b1.reshape(1, -1),
                W2, b2.reshape(1, -1))
